# Initial kernel scaffold; baseline (speedup 1.0000x reference)
import jax
import jax.numpy as jnp
from jax.experimental import pallas as pl


def kernel(x, edge_index, W1, a_src1, a_dst1, b1, W2, a_src2, a_dst2, b2, W3, a_src3, a_dst3, b3):
    raise NotImplementedError("write your pallas kernel here")


# speedup vs baseline: 3.0643x; 3.0643x over previous
import functools

import jax
import jax.numpy as jnp
from jax.experimental import pallas as pl

_NEG_SLOPE = 0.2
_BLOCK_N = 2000


def _elu(z):
    return jnp.where(z > 0, z, jnp.exp(jnp.minimum(z, 0.0)) - 1.0)


def _mm_kernel(x_ref, w_ref, asel_ref, h_ref, aa_ref):
    h = jnp.dot(x_ref[...], w_ref[...], preferred_element_type=jnp.float32)
    h_ref[...] = h
    aa_ref[...] = jnp.dot(h, asel_ref[...], preferred_element_type=jnp.float32)


def _act_mm_kernel(x_ref, b_ref, w_ref, asel_ref, h_ref, aa_ref):
    z = _elu(x_ref[...] + b_ref[...])
    h = jnp.dot(z, w_ref[...], preferred_element_type=jnp.float32)
    h_ref[...] = h
    aa_ref[...] = jnp.dot(h, asel_ref[...], preferred_element_type=jnp.float32)


def _linear_and_alphas(x, W, asel, b=None):
    n, f_in = x.shape
    ho = W.shape[1]
    k2 = asel.shape[1]
    grid = (n // _BLOCK_N,)
    out_shape = [
        jax.ShapeDtypeStruct((n, ho), jnp.float32),
        jax.ShapeDtypeStruct((n, k2), jnp.float32),
    ]
    out_specs = [
        pl.BlockSpec((_BLOCK_N, ho), lambda i: (i, 0)),
        pl.BlockSpec((_BLOCK_N, k2), lambda i: (i, 0)),
    ]
    if b is None:
        return pl.pallas_call(
            _mm_kernel,
            grid=grid,
            in_specs=[
                pl.BlockSpec((_BLOCK_N, f_in), lambda i: (i, 0)),
                pl.BlockSpec((f_in, ho), lambda i: (0, 0)),
                pl.BlockSpec((ho, k2), lambda i: (0, 0)),
            ],
            out_specs=out_specs,
            out_shape=out_shape,
        )(x, W, asel)
    return pl.pallas_call(
        _act_mm_kernel,
        grid=grid,
        in_specs=[
            pl.BlockSpec((_BLOCK_N, f_in), lambda i: (i, 0)),
            pl.BlockSpec((1, f_in), lambda i: (0, 0)),
            pl.BlockSpec((f_in, ho), lambda i: (0, 0)),
            pl.BlockSpec((ho, k2), lambda i: (0, 0)),
        ],
        out_specs=out_specs,
        out_shape=out_shape,
    )(x, b.reshape(1, f_in), W, asel)


def _logsoftmax_kernel(x_ref, b_ref, o_ref):
    z = _elu(x_ref[...] + b_ref[...])
    m = jnp.max(z, axis=1, keepdims=True)
    s = jnp.log(jnp.sum(jnp.exp(z - m), axis=1, keepdims=True))
    o_ref[...] = z - m - s


def _bias_elu_logsoftmax(x, b):
    n, c = x.shape
    return pl.pallas_call(
        _logsoftmax_kernel,
        grid=(n // _BLOCK_N,),
        in_specs=[
            pl.BlockSpec((_BLOCK_N, c), lambda i: (i, 0)),
            pl.BlockSpec((1, c), lambda i: (0, 0)),
        ],
        out_specs=pl.BlockSpec((_BLOCK_N, c), lambda i: (i, 0)),
        out_shape=jax.ShapeDtypeStruct((n, c), jnp.float32),
    )(x, b.reshape(1, c))


def _edge_aggregate(h, aa, src, dst, n, heads, outc):
    as_, ad_ = aa[:, :heads], aa[:, heads:]
    e = as_[src] + ad_[dst]
    e = jnp.where(e >= 0, e, _NEG_SLOPE * e)
    emax = jax.ops.segment_max(e, dst, num_segments=n)
    emax = jnp.where(jnp.isfinite(emax), emax, 0.0)
    ee = jnp.exp(e - emax[dst])
    denom = jax.ops.segment_sum(ee, dst, num_segments=n)
    alpha = ee / (denom[dst] + 1e-16)
    msg = h.reshape(n, heads, outc)[src] * alpha[:, :, None]
    out = jax.ops.segment_sum(msg.reshape(-1, heads * outc), dst, num_segments=n)
    return out


def _head_selector(a):
    return jax.scipy.linalg.block_diag(*(a[i][:, None] for i in range(a.shape[0])))


@jax.jit
def kernel(x, edge_index, W1, a_src1, a_dst1, b1, W2, a_src2, a_dst2, b2,
           W3, a_src3, a_dst3, b3):
    n = x.shape[0]
    loop = jnp.arange(n, dtype=edge_index.dtype)
    src = jnp.concatenate([edge_index[0], loop])
    dst = jnp.concatenate([edge_index[1], loop])

    sel1 = jnp.concatenate([_head_selector(a_src1), _head_selector(a_dst1)], axis=1)
    sel2 = jnp.concatenate([_head_selector(a_src2), _head_selector(a_dst2)], axis=1)
    sel3 = jnp.concatenate([_head_selector(a_src3), _head_selector(a_dst3)], axis=1)

    h1, aa1 = _linear_and_alphas(x, W1, sel1)
    o1 = _edge_aggregate(h1, aa1, src, dst, n, 4, 256)

    h2, aa2 = _linear_and_alphas(o1, W2, sel2, b=b1)
    o2 = _edge_aggregate(h2, aa2, src, dst, n, 4, 256)

    h3, aa3 = _linear_and_alphas(o2, W3, sel3, b=b2)
    o3 = _edge_aggregate(h3, aa3, src, dst, n, 1, W3.shape[1])

    return _bias_elu_logsoftmax(o3, b3)
